# Initial kernel scaffold; baseline (speedup 1.0000x reference)
import functools

import jax
import jax.numpy as jnp
from jax import lax
from jax.experimental import pallas as pl
from jax.experimental.pallas import tpu as pltpu
from jax.experimental.pallas import tpu_sc as plsc

NN = 100000
TT = 12
FF = 9
HID = 32
NT = 16
NC = 2
DESC = 128
JB = 8
HHALF = 16
N_ACC = NN + 16
BN = 2000
FP = jnp.float32


def _mesh():
    return plsc.VectorSubcoreMesh(core_axis_name="c", subcore_axis_name="s")


def _sc_degree(dst_r, zeros_h, ones_h):
    d_pt = dst_r.shape[1]
    nch = d_pt // JB
    rpt = NN // NT
    half = NN // NC
    hpt = half // NT

    @functools.partial(
        pl.kernel,
        out_type=jax.ShapeDtypeStruct((NN, HHALF), FP),
        mesh=_mesh(),
        scratch_types=[
            pltpu.VMEM_SHARED((N_ACC, HHALF), FP),
            pltpu.VMEM((JB, DESC), jnp.int32),
            pltpu.VMEM((DESC, HHALF), FP),
        ],
    )
    def k(dst_hbm, zeros_hbm, ones_hbm, out_hbm, acc, dst_v, ones_v):
        cid = lax.axis_index("c")
        tid = lax.axis_index("s")
        pltpu.sync_copy(zeros_hbm, acc.at[pl.ds(tid * rpt, rpt)])
        pltpu.sync_copy(ones_hbm, ones_v)
        plsc.subcore_barrier()

        def chunk(ci, carry):
            pltpu.sync_copy(dst_hbm.at[tid, pl.ds(ci * JB, JB)], dst_v)
            for j in range(JB):
                pltpu.sync_copy(ones_v, acc.at[dst_v.at[j]], add=True)
            return carry

        lax.fori_loop(0, nch, chunk, 0)
        plsc.subcore_barrier()
        off = cid * half + tid * hpt
        pltpu.sync_copy(acc.at[pl.ds(off, hpt)], out_hbm.at[pl.ds(off, hpt)])

    return k(dst_r, zeros_h, ones_h)


def _sc_propagate(z_flat, src2, dst_r):
    d_pt = dst_r.shape[1]
    nch = d_pt // JB
    rpt = NN // NT

    @functools.partial(
        pl.kernel,
        out_type=jax.ShapeDtypeStruct((NC, NN, HHALF), FP),
        mesh=_mesh(),
        scratch_types=[
            pltpu.VMEM_SHARED((N_ACC, HHALF), FP),
            pltpu.VMEM((JB, DESC), jnp.int32),
            pltpu.VMEM((JB, DESC), jnp.int32),
            pltpu.VMEM((JB, DESC, HHALF), FP),
            pltpu.SemaphoreType.DMA,
        ],
    )
    def k(z_hbm, src_hbm, dst_hbm, out_hbm, acc, src_v, dst_v, rows_v, sem):
        cid = lax.axis_index("c")
        tid = lax.axis_index("s")
        pltpu.sync_copy(
            z_hbm.at[pl.ds(cid * NN + tid * rpt, rpt)],
            acc.at[pl.ds(tid * rpt, rpt)],
        )
        plsc.subcore_barrier()

        def chunk(ci, carry):
            pltpu.sync_copy(src_hbm.at[cid, tid, pl.ds(ci * JB, JB)], src_v)
            pltpu.sync_copy(dst_hbm.at[tid, pl.ds(ci * JB, JB)], dst_v)
            cps = [
                pltpu.async_copy(z_hbm.at[src_v.at[j]], rows_v.at[j], sem)
                for j in range(JB)
            ]
            for cp in cps:
                cp.wait()
            for j in range(JB):
                pltpu.sync_copy(rows_v.at[j], acc.at[dst_v.at[j]], add=True)
            return carry

        lax.fori_loop(0, nch, chunk, 0)
        plsc.subcore_barrier()
        pltpu.sync_copy(
            acc.at[pl.ds(tid * rpt, rpt)],
            out_hbm.at[cid, pl.ds(tid * rpt, rpt)],
        )

    return k(z_flat, src2, dst_r)


def _tc_lstm(x2, deg16, wrep, whh_t, bsum, w1):
    grid = (NN // BN,)

    def body(x_ref, deg_ref, wrep_ref, whh_ref, b_ref, w1_ref, z_ref, dis_ref):
        x = x_ref[...]
        g_all = jnp.dot(x, wrep_ref[...], preferred_element_type=FP,
                        precision=lax.Precision.HIGHEST)
        whh = whh_ref[...]
        b = b_ref[...]
        h = jnp.zeros((BN, HID), dtype=FP)
        c = jnp.zeros((BN, HID), dtype=FP)
        for t in range(TT):
            g = (g_all[:, t * 128:(t + 1) * 128] + b
                 + jnp.dot(h, whh, preferred_element_type=FP,
                           precision=lax.Precision.HIGHEST))
            ig = jax.nn.sigmoid(g[:, 0:32])
            fg = jax.nn.sigmoid(g[:, 32:64])
            gg = jnp.tanh(g[:, 64:96])
            og = jax.nn.sigmoid(g[:, 96:128])
            c = fg * c + ig * gg
            h = og * jnp.tanh(c)
        dis = lax.rsqrt(deg_ref[:, 0:1] + 1.0)
        z = dis * jnp.dot(h, w1_ref[...], preferred_element_type=FP,
                          precision=lax.Precision.HIGHEST)
        z_ref[0] = z[:, :HHALF]
        z_ref[1] = z[:, HHALF:]
        dis_ref[...] = dis

    return pl.pallas_call(
        body,
        grid=grid,
        in_specs=[
            pl.BlockSpec((BN, TT * FF), lambda i: (i, 0)),
            pl.BlockSpec((BN, HHALF), lambda i: (i, 0)),
            pl.BlockSpec((TT * FF, TT * 128), lambda i: (0, 0)),
            pl.BlockSpec((HID, 128), lambda i: (0, 0)),
            pl.BlockSpec((1, 128), lambda i: (0, 0)),
            pl.BlockSpec((HID, HID), lambda i: (0, 0)),
        ],
        out_specs=[
            pl.BlockSpec((NC, BN, HHALF), lambda i: (0, i, 0)),
            pl.BlockSpec((BN, 1), lambda i: (i, 0)),
        ],
        out_shape=[
            jax.ShapeDtypeStruct((NC, NN, HHALF), FP),
            jax.ShapeDtypeStruct((NN, 1), FP),
        ],
    )(x2, deg16, wrep, whh_t, bsum, w1)


def _tc_glue(acc, dis, b, w):
    grid = (NN // BN,)

    def body(acc_ref, dis_ref, b_ref, w_ref, z_ref):
        a = jnp.concatenate([acc_ref[0], acc_ref[1]], axis=1)
        dis = dis_ref[...]
        hcur = jnp.maximum(a * dis + b_ref[...], 0.0)
        z = dis * jnp.dot(hcur, w_ref[...], preferred_element_type=FP,
                          precision=lax.Precision.HIGHEST)
        z_ref[0] = z[:, :HHALF]
        z_ref[1] = z[:, HHALF:]

    return pl.pallas_call(
        body,
        grid=grid,
        in_specs=[
            pl.BlockSpec((NC, BN, HHALF), lambda i: (0, i, 0)),
            pl.BlockSpec((BN, 1), lambda i: (i, 0)),
            pl.BlockSpec((1, HID), lambda i: (0, 0)),
            pl.BlockSpec((HID, HID), lambda i: (0, 0)),
        ],
        out_specs=pl.BlockSpec((NC, BN, HHALF), lambda i: (0, i, 0)),
        out_shape=jax.ShapeDtypeStruct((NC, NN, HHALF), FP),
    )(acc, dis, b, w)


def _tc_head(acc, dis, b3, wh1, bh1, wh2row, bh2):
    grid = (NN // BN,)

    def body(acc_ref, dis_ref, b3_ref, wh1_ref, bh1_ref, wh2_ref, bh2_ref, o_ref):
        a = jnp.concatenate([acc_ref[0], acc_ref[1]], axis=1)
        h3 = a * dis_ref[...] + b3_ref[...]
        hid = jnp.maximum(
            jnp.dot(h3, wh1_ref[...], preferred_element_type=FP,
                    precision=lax.Precision.HIGHEST) + bh1_ref[...], 0.0)
        o = jnp.sum(hid * wh2_ref[...], axis=1, keepdims=True) + bh2_ref[...]
        o_ref[...] = jax.nn.sigmoid(o)

    return pl.pallas_call(
        body,
        grid=grid,
        in_specs=[
            pl.BlockSpec((NC, BN, HHALF), lambda i: (0, i, 0)),
            pl.BlockSpec((BN, 1), lambda i: (i, 0)),
            pl.BlockSpec((1, HID), lambda i: (0, 0)),
            pl.BlockSpec((HID, HHALF), lambda i: (0, 0)),
            pl.BlockSpec((1, HHALF), lambda i: (0, 0)),
            pl.BlockSpec((1, HHALF), lambda i: (0, 0)),
            pl.BlockSpec((1, 1), lambda i: (0, 0)),
        ],
        out_specs=pl.BlockSpec((BN, 1), lambda i: (i, 0)),
        out_shape=jax.ShapeDtypeStruct((NN, 1), FP),
    )(acc, dis, b3, wh1, bh1, wh2row, bh2)


def kernel(x_seq, edge_index, W_ih, W_hh, b_ih, b_hh, W1, b1, W2, b2, W3, b3,
           Wh1, bh1, Wh2, bh2):
    e = edge_index.shape[1]
    d_pt = -(-e // (NT * DESC))
    d_pt = -(-d_pt // JB) * JB
    e_pad = NT * d_pt * DESC
    pad = e_pad - e
    src = jnp.concatenate([edge_index[0], jnp.zeros((pad,), jnp.int32)])
    dst = jnp.concatenate([edge_index[1], jnp.full((pad,), NN, jnp.int32)])
    dst_r = dst.reshape(NT, d_pt, DESC)
    src_r = src.reshape(NT, d_pt, DESC)
    src2 = jnp.stack([src_r, src_r + NN])

    wrep = jnp.zeros((TT * FF, TT * 128), FP)
    wih_t = W_ih.T
    for t in range(TT):
        wrep = wrep.at[t * FF:(t + 1) * FF, t * 128:(t + 1) * 128].set(wih_t)
    whh_t = W_hh.T
    bsum = (b_ih + b_hh).reshape(1, 128)
    x2 = x_seq.reshape(NN, TT * FF)

    zeros_h = jnp.zeros((NN // NT, HHALF), FP)
    ones_h = jnp.ones((DESC, HHALF), FP)

    deg16 = _sc_degree(dst_r, zeros_h, ones_h)
    z1, dis = _tc_lstm(x2, deg16, wrep, whh_t, bsum, W1)

    acc1 = _sc_propagate(z1.reshape(NC * NN, HHALF), src2, dst_r)
    z2 = _tc_glue(acc1, dis, b1.reshape(1, HID), W2)
    acc2 = _sc_propagate(z2.reshape(NC * NN, HHALF), src2, dst_r)
    z3 = _tc_glue(acc2, dis, b2.reshape(1, HID), W3)
    acc3 = _sc_propagate(z3.reshape(NC * NN, HHALF), src2, dst_r)

    out = _tc_head(acc3, dis, b3.reshape(1, HID), Wh1, bh1.reshape(1, HHALF),
                   Wh2.reshape(1, HHALF), bh2.reshape(1, 1))
    return out.reshape(NN)


# speedup vs baseline: 12.2895x; 12.2895x over previous
import functools

import jax
import jax.numpy as jnp
from jax import lax
from jax.experimental import pallas as pl
from jax.experimental.pallas import tpu as pltpu
from jax.experimental.pallas import tpu_sc as plsc

NN = 100000
TT = 12
FF = 9
HID = 32
NT = 16
NC = 2
DESC = 128
JB = 8
HHALF = 16
N_ACC = NN + 16
BN = 2000
FP = jnp.float32


def _mesh():
    return plsc.VectorSubcoreMesh(core_axis_name="c", subcore_axis_name="s")


def _tile_rows(r):
    per = (r // NT) // 8 * 8
    rem = r - NT * per
    assert rem % 8 == 0
    return per, rem


def _sc_degree(dst_r, zeros_h, ones_h):
    d_pt = dst_r.shape[1]
    nch = d_pt // JB
    zper, zrem = _tile_rows(NN)
    half = NN // NC
    hper, hrem = _tile_rows(half)

    @functools.partial(
        pl.kernel,
        out_type=jax.ShapeDtypeStruct((NN, HHALF), FP),
        mesh=_mesh(),
        compiler_params=pltpu.CompilerParams(use_tc_tiling_on_sc=False),
        scratch_types=[
            pltpu.VMEM_SHARED((N_ACC, HHALF), FP),
            pltpu.VMEM((JB, DESC), jnp.int32),
            pltpu.VMEM((DESC, HHALF), FP),
        ],
    )
    def k(dst_hbm, zeros_hbm, ones_hbm, out_hbm, acc, dst_v, ones_v):
        cid = lax.axis_index("c")
        tid = lax.axis_index("s")
        pltpu.sync_copy(zeros_hbm.at[pl.ds(0, zper)],
                        acc.at[pl.ds(tid * zper, zper)])

        @pl.when(tid == 0)
        def _():
            pltpu.sync_copy(zeros_hbm.at[pl.ds(0, zrem)],
                            acc.at[pl.ds(NT * zper, zrem)])

        pltpu.sync_copy(ones_hbm, ones_v)
        plsc.subcore_barrier()

        def chunk(ci, carry):
            pltpu.sync_copy(dst_hbm.at[tid, pl.ds(ci * JB, JB)], dst_v)
            for j in range(JB):
                pltpu.sync_copy(ones_v, acc.at[dst_v.at[j]], add=True)
            return carry

        lax.fori_loop(0, nch, chunk, 0)
        plsc.subcore_barrier()
        off = cid * half + tid * hper
        pltpu.sync_copy(acc.at[pl.ds(off, hper)], out_hbm.at[pl.ds(off, hper)])

        @pl.when(tid == 0)
        def _():
            off2 = cid * half + NT * hper
            pltpu.sync_copy(acc.at[pl.ds(off2, hrem)],
                            out_hbm.at[pl.ds(off2, hrem)])

    return k(dst_r, zeros_h, ones_h)


def _sc_propagate(z_flat, src2, dst_r):
    d_pt = dst_r.shape[1]
    nch = d_pt // JB
    per, rem = _tile_rows(NN)

    @functools.partial(
        pl.kernel,
        out_type=jax.ShapeDtypeStruct((NC, NN, HHALF), FP),
        mesh=_mesh(),
        compiler_params=pltpu.CompilerParams(use_tc_tiling_on_sc=False),
        scratch_types=[
            pltpu.VMEM_SHARED((N_ACC, HHALF), FP),
            pltpu.VMEM((JB, DESC), jnp.int32),
            pltpu.VMEM((JB, DESC), jnp.int32),
            pltpu.VMEM((JB, DESC, HHALF), FP),
            pltpu.SemaphoreType.DMA,
        ],
    )
    def k(z_hbm, src_hbm, dst_hbm, out_hbm, acc, src_v, dst_v, rows_v, sem):
        cid = lax.axis_index("c")
        tid = lax.axis_index("s")
        pltpu.sync_copy(
            z_hbm.at[pl.ds(cid * NN + tid * per, per)],
            acc.at[pl.ds(tid * per, per)],
        )

        @pl.when(tid == 0)
        def _():
            pltpu.sync_copy(
                z_hbm.at[pl.ds(cid * NN + NT * per, rem)],
                acc.at[pl.ds(NT * per, rem)],
            )

        plsc.subcore_barrier()

        def chunk(ci, carry):
            pltpu.sync_copy(src_hbm.at[cid, tid, pl.ds(ci * JB, JB)], src_v)
            pltpu.sync_copy(dst_hbm.at[tid, pl.ds(ci * JB, JB)], dst_v)
            cps = [
                pltpu.async_copy(z_hbm.at[src_v.at[j]], rows_v.at[j], sem)
                for j in range(JB)
            ]
            for cp in cps:
                cp.wait()
            for j in range(JB):
                pltpu.sync_copy(rows_v.at[j], acc.at[dst_v.at[j]], add=True)
            return carry

        lax.fori_loop(0, nch, chunk, 0)
        plsc.subcore_barrier()
        pltpu.sync_copy(
            acc.at[pl.ds(tid * per, per)],
            out_hbm.at[cid, pl.ds(tid * per, per)],
        )

        @pl.when(tid == 0)
        def _():
            pltpu.sync_copy(
                acc.at[pl.ds(NT * per, rem)],
                out_hbm.at[cid, pl.ds(NT * per, rem)],
            )

    return k(z_flat, src2, dst_r)


def _tc_lstm(x2, deg16, wrep, whh_t, bsum, w1):
    grid = (NN // BN,)

    def body(x_ref, deg_ref, wrep_ref, whh_ref, b_ref, w1_ref, z_ref, dis_ref):
        x = x_ref[...]
        g_all = jnp.dot(x, wrep_ref[...], preferred_element_type=FP,
                        precision=lax.Precision.HIGHEST)
        whh = whh_ref[...]
        b = b_ref[...]
        h = jnp.zeros((BN, HID), dtype=FP)
        c = jnp.zeros((BN, HID), dtype=FP)
        for t in range(TT):
            g = (g_all[:, t * 128:(t + 1) * 128] + b
                 + jnp.dot(h, whh, preferred_element_type=FP,
                           precision=lax.Precision.HIGHEST))
            ig = jax.nn.sigmoid(g[:, 0:32])
            fg = jax.nn.sigmoid(g[:, 32:64])
            gg = jnp.tanh(g[:, 64:96])
            og = jax.nn.sigmoid(g[:, 96:128])
            c = fg * c + ig * gg
            h = og * jnp.tanh(c)
        dis = lax.rsqrt(deg_ref[:, 0:1] + 1.0)
        z = dis * jnp.dot(h, w1_ref[...], preferred_element_type=FP,
                          precision=lax.Precision.HIGHEST)
        z_ref[0] = z[:, :HHALF]
        z_ref[1] = z[:, HHALF:]
        dis_ref[...] = dis

    return pl.pallas_call(
        body,
        grid=grid,
        in_specs=[
            pl.BlockSpec((BN, TT * FF), lambda i: (i, 0)),
            pl.BlockSpec((BN, HHALF), lambda i: (i, 0)),
            pl.BlockSpec((TT * FF, TT * 128), lambda i: (0, 0)),
            pl.BlockSpec((HID, 128), lambda i: (0, 0)),
            pl.BlockSpec((1, 128), lambda i: (0, 0)),
            pl.BlockSpec((HID, HID), lambda i: (0, 0)),
        ],
        out_specs=[
            pl.BlockSpec((NC, BN, HHALF), lambda i: (0, i, 0)),
            pl.BlockSpec((BN, 1), lambda i: (i, 0)),
        ],
        out_shape=[
            jax.ShapeDtypeStruct((NC, NN, HHALF), FP),
            jax.ShapeDtypeStruct((NN, 1), FP),
        ],
    )(x2, deg16, wrep, whh_t, bsum, w1)


def _tc_glue(acc, dis, b, w):
    grid = (NN // BN,)

    def body(acc_ref, dis_ref, b_ref, w_ref, z_ref):
        a = jnp.concatenate([acc_ref[0], acc_ref[1]], axis=1)
        dis = dis_ref[...]
        hcur = jnp.maximum(a * dis + b_ref[...], 0.0)
        z = dis * jnp.dot(hcur, w_ref[...], preferred_element_type=FP,
                          precision=lax.Precision.HIGHEST)
        z_ref[0] = z[:, :HHALF]
        z_ref[1] = z[:, HHALF:]

    return pl.pallas_call(
        body,
        grid=grid,
        in_specs=[
            pl.BlockSpec((NC, BN, HHALF), lambda i: (0, i, 0)),
            pl.BlockSpec((BN, 1), lambda i: (i, 0)),
            pl.BlockSpec((1, HID), lambda i: (0, 0)),
            pl.BlockSpec((HID, HID), lambda i: (0, 0)),
        ],
        out_specs=pl.BlockSpec((NC, BN, HHALF), lambda i: (0, i, 0)),
        out_shape=jax.ShapeDtypeStruct((NC, NN, HHALF), FP),
    )(acc, dis, b, w)


def _tc_head(acc, dis, b3, wh1, bh1, wh2row, bh2):
    grid = (NN // BN,)

    def body(acc_ref, dis_ref, b3_ref, wh1_ref, bh1_ref, wh2_ref, bh2_ref, o_ref):
        a = jnp.concatenate([acc_ref[0], acc_ref[1]], axis=1)
        h3 = a * dis_ref[...] + b3_ref[...]
        hid = jnp.maximum(
            jnp.dot(h3, wh1_ref[...], preferred_element_type=FP,
                    precision=lax.Precision.HIGHEST) + bh1_ref[...], 0.0)
        o = jnp.sum(hid * wh2_ref[...], axis=1, keepdims=True) + bh2_ref[...]
        o_ref[...] = jax.nn.sigmoid(o)

    return pl.pallas_call(
        body,
        grid=grid,
        in_specs=[
            pl.BlockSpec((NC, BN, HHALF), lambda i: (0, i, 0)),
            pl.BlockSpec((BN, 1), lambda i: (i, 0)),
            pl.BlockSpec((1, HID), lambda i: (0, 0)),
            pl.BlockSpec((HID, HHALF), lambda i: (0, 0)),
            pl.BlockSpec((1, HHALF), lambda i: (0, 0)),
            pl.BlockSpec((1, HHALF), lambda i: (0, 0)),
            pl.BlockSpec((1, 1), lambda i: (0, 0)),
        ],
        out_specs=pl.BlockSpec((BN, 1), lambda i: (i, 0)),
        out_shape=jax.ShapeDtypeStruct((NN, 1), FP),
    )(acc, dis, b3, wh1, bh1, wh2row, bh2)


def kernel(x_seq, edge_index, W_ih, W_hh, b_ih, b_hh, W1, b1, W2, b2, W3, b3,
           Wh1, bh1, Wh2, bh2):
    e = edge_index.shape[1]
    d_pt = -(-e // (NT * DESC))
    d_pt = -(-d_pt // JB) * JB
    e_pad = NT * d_pt * DESC
    pad = e_pad - e
    src = jnp.concatenate([edge_index[0], jnp.zeros((pad,), jnp.int32)])
    dst = jnp.concatenate([edge_index[1], jnp.full((pad,), NN, jnp.int32)])
    dst_r = dst.reshape(NT, d_pt, DESC)
    src_r = src.reshape(NT, d_pt, DESC)
    src2 = jnp.stack([src_r, src_r + NN])

    wrep = jnp.zeros((TT * FF, TT * 128), FP)
    wih_t = W_ih.T
    for t in range(TT):
        wrep = wrep.at[t * FF:(t + 1) * FF, t * 128:(t + 1) * 128].set(wih_t)
    whh_t = W_hh.T
    bsum = (b_ih + b_hh).reshape(1, 128)
    x2 = x_seq.reshape(NN, TT * FF)

    zeros_h = jnp.zeros((NN // NT, HHALF), FP)
    ones_h = jnp.ones((DESC, HHALF), FP)

    deg16 = _sc_degree(dst_r, zeros_h, ones_h)
    z1, dis = _tc_lstm(x2, deg16, wrep, whh_t, bsum, W1)

    acc1 = _sc_propagate(z1.reshape(NC * NN, HHALF), src2, dst_r)
    z2 = _tc_glue(acc1, dis, b1.reshape(1, HID), W2)
    acc2 = _sc_propagate(z2.reshape(NC * NN, HHALF), src2, dst_r)
    z3 = _tc_glue(acc2, dis, b2.reshape(1, HID), W3)
    acc3 = _sc_propagate(z3.reshape(NC * NN, HHALF), src2, dst_r)

    out = _tc_head(acc3, dis, b3.reshape(1, HID), Wh1, bh1.reshape(1, HHALF),
                   Wh2.reshape(1, HHALF), bh2.reshape(1, 1))
    return out.reshape(NN)


# speedup vs baseline: 13.0839x; 1.0646x over previous
import functools

import jax
import jax.numpy as jnp
from jax import lax
from jax.experimental import pallas as pl
from jax.experimental.pallas import tpu as pltpu
from jax.experimental.pallas import tpu_sc as plsc

NN = 100000
TT = 12
FF = 9
HID = 32
NT = 16
NC = 2
DESC = 128
JB = 6
HHALF = 16
N_ACC = NN + 8
BN = 2000
FP = jnp.float32


def _mesh():
    return plsc.VectorSubcoreMesh(core_axis_name="c", subcore_axis_name="s")


def _tile_rows(r):
    per = (r // NT) // 8 * 8
    rem = r - NT * per
    assert rem % 8 == 0
    return per, rem


def _sc_degree(dst_r, zeros_h, ones_h):
    d_pt = dst_r.shape[1]
    half_pt = d_pt // NC
    nch = half_pt // JB
    zper, zrem = _tile_rows(NN)

    @functools.partial(
        pl.kernel,
        out_type=jax.ShapeDtypeStruct((NC, NN, HHALF), FP),
        mesh=_mesh(),
        compiler_params=pltpu.CompilerParams(use_tc_tiling_on_sc=False),
        scratch_types=[
            pltpu.VMEM_SHARED((N_ACC, HHALF), FP),
            pltpu.VMEM((JB, DESC), jnp.int32),
            pltpu.VMEM((DESC, HHALF), FP),
        ],
    )
    def k(dst_hbm, zeros_hbm, ones_hbm, out_hbm, acc, dst_v, ones_v):
        cid = lax.axis_index("c")
        tid = lax.axis_index("s")
        pltpu.sync_copy(zeros_hbm.at[pl.ds(0, zper)],
                        acc.at[pl.ds(tid * zper, zper)])

        @pl.when(tid == 0)
        def _():
            pltpu.sync_copy(zeros_hbm.at[pl.ds(0, zrem)],
                            acc.at[pl.ds(NT * zper, zrem)])

        pltpu.sync_copy(ones_hbm, ones_v)
        plsc.subcore_barrier()
        base = cid * half_pt

        def chunk(ci, carry):
            pltpu.sync_copy(dst_hbm.at[tid, pl.ds(base + ci * JB, JB)], dst_v)
            for j in range(JB):
                pltpu.sync_copy(ones_v, acc.at[dst_v.at[j]], add=True)
            return carry

        lax.fori_loop(0, nch, chunk, 0)
        plsc.subcore_barrier()
        pltpu.sync_copy(acc.at[pl.ds(tid * zper, zper)],
                        out_hbm.at[cid, pl.ds(tid * zper, zper)])

        @pl.when(tid == 0)
        def _():
            pltpu.sync_copy(acc.at[pl.ds(NT * zper, zrem)],
                            out_hbm.at[cid, pl.ds(NT * zper, zrem)])

    return k(dst_r, zeros_h, ones_h)


def _sc_propagate(z_flat, src_r, dst_r):
    d_pt = dst_r.shape[1]
    nch = d_pt // JB
    per, rem = _tile_rows(NN)
    ch = JB * DESC

    @functools.partial(
        pl.kernel,
        out_type=jax.ShapeDtypeStruct((NC, NN, HHALF), FP),
        mesh=_mesh(),
        compiler_params=pltpu.CompilerParams(use_tc_tiling_on_sc=False),
        scratch_types=[
            pltpu.VMEM_SHARED((N_ACC, HHALF), FP),
            pltpu.VMEM((2, JB, DESC), jnp.int32),
            pltpu.VMEM((2, JB, DESC), jnp.int32),
            pltpu.VMEM((2, ch, HHALF), FP),
            pltpu.SemaphoreType.DMA((2,)),
            pltpu.SemaphoreType.DMA((2,)),
        ],
    )
    def k(z_hbm, src_hbm, dst_hbm, out_hbm, acc, src_v, dst_v, rows_v, gsem, isem):
        cid = lax.axis_index("c")
        tid = lax.axis_index("s")
        zc = z_hbm.at[pl.ds(cid * NN, NN)]
        pltpu.sync_copy(zc.at[pl.ds(tid * per, per)], acc.at[pl.ds(tid * per, per)])

        @pl.when(tid == 0)
        def _():
            pltpu.sync_copy(zc.at[pl.ds(NT * per, rem)], acc.at[pl.ds(NT * per, rem)])

        plsc.subcore_barrier()

        def loadidx(c, b):
            pltpu.async_copy(src_hbm.at[tid, pl.ds(c * JB, JB)],
                             src_v.at[b], isem.at[b])
            pltpu.async_copy(dst_hbm.at[tid, pl.ds(c * JB, JB)],
                             dst_v.at[b], isem.at[b])

        def wait_idx(b):
            pltpu.make_async_copy(src_hbm.at[0, pl.ds(0, JB)],
                                  src_v.at[b], isem.at[b]).wait()
            pltpu.make_async_copy(dst_hbm.at[0, pl.ds(0, JB)],
                                  dst_v.at[b], isem.at[b]).wait()

        def fire(b):
            for j in range(JB):
                pltpu.async_copy(zc.at[src_v.at[b, j]],
                                 rows_v.at[b, pl.ds(j * DESC, DESC)], gsem.at[b])

        def drain(b):
            pltpu.make_async_copy(z_hbm.at[pl.ds(0, ch)], rows_v.at[b],
                                  gsem.at[b]).wait()

        def scat(b):
            for j in range(JB):
                pltpu.sync_copy(rows_v.at[b, pl.ds(j * DESC, DESC)],
                                acc.at[dst_v.at[b, j]], add=True)

        loadidx(0, 0)
        loadidx(1, 1)
        wait_idx(0)
        fire(0)

        def body(c, carry):
            b = lax.rem(c, 2)
            pb = lax.rem(c + 1, 2)
            wait_idx(b)
            fire(b)
            drain(pb)
            scat(pb)

            @pl.when(c < nch - 1)
            def _():
                loadidx(c + 1, pb)

            return carry

        lax.fori_loop(1, nch, body, 0)
        lastb = (nch - 1) % 2
        drain(lastb)
        scat(lastb)
        plsc.subcore_barrier()
        pltpu.sync_copy(
            acc.at[pl.ds(tid * per, per)],
            out_hbm.at[cid, pl.ds(tid * per, per)],
        )

        @pl.when(tid == 0)
        def _():
            pltpu.sync_copy(
                acc.at[pl.ds(NT * per, rem)],
                out_hbm.at[cid, pl.ds(NT * per, rem)],
            )

    return k(z_flat, src_r, dst_r)


def _tc_lstm(x2, deg16, wrep, whh_t, bsum, w1):
    grid = (NN // BN,)

    def body(x_ref, deg_ref, wrep_ref, whh_ref, b_ref, w1_ref, z_ref, dis_ref):
        x = x_ref[...]
        g_all = jnp.dot(x, wrep_ref[...], preferred_element_type=FP,
                        precision=lax.Precision.HIGHEST)
        whh = whh_ref[...]
        b = b_ref[...]
        h = jnp.zeros((BN, HID), dtype=FP)
        c = jnp.zeros((BN, HID), dtype=FP)
        for t in range(TT):
            g = (g_all[:, t * 128:(t + 1) * 128] + b
                 + jnp.dot(h, whh, preferred_element_type=FP,
                           precision=lax.Precision.HIGHEST))
            ig = jax.nn.sigmoid(g[:, 0:32])
            fg = jax.nn.sigmoid(g[:, 32:64])
            gg = jnp.tanh(g[:, 64:96])
            og = jax.nn.sigmoid(g[:, 96:128])
            c = fg * c + ig * gg
            h = og * jnp.tanh(c)
        dis = lax.rsqrt(deg_ref[0, :, 0:1] + deg_ref[1, :, 0:1] + 1.0)
        z = dis * jnp.dot(h, w1_ref[...], preferred_element_type=FP,
                          precision=lax.Precision.HIGHEST)
        z_ref[0] = z[:, :HHALF]
        z_ref[1] = z[:, HHALF:]
        dis_ref[...] = dis

    return pl.pallas_call(
        body,
        grid=grid,
        in_specs=[
            pl.BlockSpec((BN, TT * FF), lambda i: (i, 0)),
            pl.BlockSpec((NC, BN, HHALF), lambda i: (0, i, 0)),
            pl.BlockSpec((TT * FF, TT * 128), lambda i: (0, 0)),
            pl.BlockSpec((HID, 128), lambda i: (0, 0)),
            pl.BlockSpec((1, 128), lambda i: (0, 0)),
            pl.BlockSpec((HID, HID), lambda i: (0, 0)),
        ],
        out_specs=[
            pl.BlockSpec((NC, BN, HHALF), lambda i: (0, i, 0)),
            pl.BlockSpec((BN, 1), lambda i: (i, 0)),
        ],
        out_shape=[
            jax.ShapeDtypeStruct((NC, NN, HHALF), FP),
            jax.ShapeDtypeStruct((NN, 1), FP),
        ],
    )(x2, deg16, wrep, whh_t, bsum, w1)


def _tc_glue(acc, dis, b, w):
    grid = (NN // BN,)

    def body(acc_ref, dis_ref, b_ref, w_ref, z_ref):
        a = jnp.concatenate([acc_ref[0], acc_ref[1]], axis=1)
        dis = dis_ref[...]
        hcur = jnp.maximum(a * dis + b_ref[...], 0.0)
        z = dis * jnp.dot(hcur, w_ref[...], preferred_element_type=FP,
                          precision=lax.Precision.HIGHEST)
        z_ref[0] = z[:, :HHALF]
        z_ref[1] = z[:, HHALF:]

    return pl.pallas_call(
        body,
        grid=grid,
        in_specs=[
            pl.BlockSpec((NC, BN, HHALF), lambda i: (0, i, 0)),
            pl.BlockSpec((BN, 1), lambda i: (i, 0)),
            pl.BlockSpec((1, HID), lambda i: (0, 0)),
            pl.BlockSpec((HID, HID), lambda i: (0, 0)),
        ],
        out_specs=pl.BlockSpec((NC, BN, HHALF), lambda i: (0, i, 0)),
        out_shape=jax.ShapeDtypeStruct((NC, NN, HHALF), FP),
    )(acc, dis, b, w)


def _tc_head(acc, dis, b3, wh1, bh1, wh2row, bh2):
    grid = (NN // BN,)

    def body(acc_ref, dis_ref, b3_ref, wh1_ref, bh1_ref, wh2_ref, bh2_ref, o_ref):
        a = jnp.concatenate([acc_ref[0], acc_ref[1]], axis=1)
        h3 = a * dis_ref[...] + b3_ref[...]
        hid = jnp.maximum(
            jnp.dot(h3, wh1_ref[...], preferred_element_type=FP,
                    precision=lax.Precision.HIGHEST) + bh1_ref[...], 0.0)
        o = jnp.sum(hid * wh2_ref[...], axis=1, keepdims=True) + bh2_ref[...]
        o_ref[...] = jax.nn.sigmoid(o)

    return pl.pallas_call(
        body,
        grid=grid,
        in_specs=[
            pl.BlockSpec((NC, BN, HHALF), lambda i: (0, i, 0)),
            pl.BlockSpec((BN, 1), lambda i: (i, 0)),
            pl.BlockSpec((1, HID), lambda i: (0, 0)),
            pl.BlockSpec((HID, HHALF), lambda i: (0, 0)),
            pl.BlockSpec((1, HHALF), lambda i: (0, 0)),
            pl.BlockSpec((1, HHALF), lambda i: (0, 0)),
            pl.BlockSpec((1, 1), lambda i: (0, 0)),
        ],
        out_specs=pl.BlockSpec((BN, 1), lambda i: (i, 0)),
        out_shape=jax.ShapeDtypeStruct((NN, 1), FP),
    )(acc, dis, b3, wh1, bh1, wh2row, bh2)


def kernel(x_seq, edge_index, W_ih, W_hh, b_ih, b_hh, W1, b1, W2, b2, W3, b3,
           Wh1, bh1, Wh2, bh2):
    e = edge_index.shape[1]
    d_pt = -(-e // (NT * DESC))
    d_pt = -(-d_pt // (NC * JB)) * (NC * JB)
    e_pad = NT * d_pt * DESC
    pad = e_pad - e
    src = jnp.concatenate([edge_index[0], jnp.zeros((pad,), jnp.int32)])
    dst = jnp.concatenate([edge_index[1], jnp.full((pad,), NN, jnp.int32)])
    dst_r = dst.reshape(NT, d_pt, DESC)
    src_r = src.reshape(NT, d_pt, DESC)

    wrep = jnp.zeros((TT * FF, TT * 128), FP)
    wih_t = W_ih.T
    for t in range(TT):
        wrep = wrep.at[t * FF:(t + 1) * FF, t * 128:(t + 1) * 128].set(wih_t)
    whh_t = W_hh.T
    bsum = (b_ih + b_hh).reshape(1, 128)
    x2 = x_seq.reshape(NN, TT * FF)

    zeros_h = jnp.zeros((NN // NT, HHALF), FP)
    ones_h = jnp.ones((DESC, HHALF), FP)

    deg2 = _sc_degree(dst_r, zeros_h, ones_h)
    z1, dis = _tc_lstm(x2, deg2, wrep, whh_t, bsum, W1)

    acc1 = _sc_propagate(z1.reshape(NC * NN, HHALF), src_r, dst_r)
    z2 = _tc_glue(acc1, dis, b1.reshape(1, HID), W2)
    acc2 = _sc_propagate(z2.reshape(NC * NN, HHALF), src_r, dst_r)
    z3 = _tc_glue(acc2, dis, b2.reshape(1, HID), W3)
    acc3 = _sc_propagate(z3.reshape(NC * NN, HHALF), src_r, dst_r)

    out = _tc_head(acc3, dis, b3.reshape(1, HID), Wh1, bh1.reshape(1, HHALF),
                   Wh2.reshape(1, HHALF), bh2.reshape(1, 1))
    return out.reshape(NN)


# speedup vs baseline: 15.7494x; 1.2037x over previous
import functools

import jax
import jax.numpy as jnp
from jax import lax
from jax.experimental import pallas as pl
from jax.experimental.pallas import tpu as pltpu
from jax.experimental.pallas import tpu_sc as plsc

NN = 100000
TT = 12
FF = 9
HID = 32
NT = 16
NC = 2
DESC = 128
JB = 6
HHALF = 16
N_ACC = NN + 8
BN = 2000
FP = jnp.float32


def _mesh():
    return plsc.VectorSubcoreMesh(core_axis_name="c", subcore_axis_name="s")


def _tile_rows(r):
    per = (r // NT) // 8 * 8
    rem = r - NT * per
    assert rem % 8 == 0
    return per, rem


def _sc_degree(dst_r, zeros_h, ones_h):
    d_pt = dst_r.shape[1]
    half_pt = d_pt // NC
    nch = half_pt // JB
    zper, zrem = _tile_rows(NN)

    @functools.partial(
        pl.kernel,
        out_type=jax.ShapeDtypeStruct((NC, NN, HHALF), FP),
        mesh=_mesh(),
        compiler_params=pltpu.CompilerParams(use_tc_tiling_on_sc=False),
        scratch_types=[
            pltpu.VMEM_SHARED((N_ACC, HHALF), FP),
            pltpu.VMEM((JB, DESC), jnp.int32),
            pltpu.VMEM((DESC, HHALF), FP),
        ],
    )
    def k(dst_hbm, zeros_hbm, ones_hbm, out_hbm, acc, dst_v, ones_v):
        cid = lax.axis_index("c")
        tid = lax.axis_index("s")
        pltpu.sync_copy(zeros_hbm.at[pl.ds(0, zper)],
                        acc.at[pl.ds(tid * zper, zper)])

        @pl.when(tid == 0)
        def _():
            pltpu.sync_copy(zeros_hbm.at[pl.ds(0, zrem)],
                            acc.at[pl.ds(NT * zper, zrem)])

        pltpu.sync_copy(ones_hbm, ones_v)
        plsc.subcore_barrier()
        base = cid * half_pt

        def chunk(ci, carry):
            pltpu.sync_copy(dst_hbm.at[tid, pl.ds(base + ci * JB, JB)], dst_v)
            for j in range(JB):
                pltpu.sync_copy(ones_v, acc.at[dst_v.at[j]], add=True)
            return carry

        lax.fori_loop(0, nch, chunk, 0)
        plsc.subcore_barrier()
        pltpu.sync_copy(acc.at[pl.ds(tid * zper, zper)],
                        out_hbm.at[cid, pl.ds(tid * zper, zper)])

        @pl.when(tid == 0)
        def _():
            pltpu.sync_copy(acc.at[pl.ds(NT * zper, zrem)],
                            out_hbm.at[cid, pl.ds(NT * zper, zrem)])

    return k(dst_r, zeros_h, ones_h)


def _sc_propagate(z_flat, src_r, dst_r):
    d_pt = dst_r.shape[1]
    nch = d_pt // JB
    per, rem = _tile_rows(NN)
    ch = JB * DESC

    @functools.partial(
        pl.kernel,
        out_type=jax.ShapeDtypeStruct((NC, NN, HHALF), FP),
        mesh=_mesh(),
        compiler_params=pltpu.CompilerParams(use_tc_tiling_on_sc=False),
        scratch_types=[
            pltpu.VMEM_SHARED((N_ACC, HHALF), FP),
            pltpu.VMEM((4, JB, DESC), jnp.int32),
            pltpu.VMEM((4, JB, DESC), jnp.int32),
            pltpu.VMEM((2, ch, HHALF), FP),
            pltpu.SemaphoreType.DMA((2,)),
            pltpu.SemaphoreType.DMA((4,)),
            pltpu.SemaphoreType.DMA((2,)),
        ],
    )
    def k(z_hbm, src_hbm, dst_hbm, out_hbm, acc, src_v, dst_v, rows_v,
          gsem, isem, ssem):
        cid = lax.axis_index("c")
        tid = lax.axis_index("s")
        zc = z_hbm.at[pl.ds(cid * NN, NN)]
        pltpu.sync_copy(zc.at[pl.ds(tid * per, per)], acc.at[pl.ds(tid * per, per)])

        @pl.when(tid == 0)
        def _():
            pltpu.sync_copy(zc.at[pl.ds(NT * per, rem)], acc.at[pl.ds(NT * per, rem)])

        plsc.subcore_barrier()

        def loadidx(c, b4):
            pltpu.async_copy(src_hbm.at[tid, pl.ds(c * JB, JB)],
                             src_v.at[b4], isem.at[b4])
            pltpu.async_copy(dst_hbm.at[tid, pl.ds(c * JB, JB)],
                             dst_v.at[b4], isem.at[b4])

        def wait_idx(b4):
            pltpu.make_async_copy(src_hbm.at[0, pl.ds(0, JB)],
                                  src_v.at[b4], isem.at[b4]).wait()
            pltpu.make_async_copy(dst_hbm.at[0, pl.ds(0, JB)],
                                  dst_v.at[b4], isem.at[b4]).wait()

        def fire_g(b4, b):
            for j in range(JB):
                pltpu.async_copy(zc.at[src_v.at[b4, j]],
                                 rows_v.at[b, pl.ds(j * DESC, DESC)], gsem.at[b])

        def drain_g(b):
            pltpu.make_async_copy(z_hbm.at[pl.ds(0, ch)], rows_v.at[b],
                                  gsem.at[b]).wait()

        def fire_s(b4, b):
            for j in range(JB):
                pltpu.async_copy(rows_v.at[b, pl.ds(j * DESC, DESC)],
                                 acc.at[dst_v.at[b4, j]], ssem.at[b], add=True)

        def drain_s(b):
            pltpu.make_async_copy(rows_v.at[b], acc.at[pl.ds(0, ch)],
                                  ssem.at[b]).wait()

        loadidx(0, 0)
        loadidx(1, 1)
        loadidx(2, 2)
        wait_idx(0)
        fire_g(0, 0)
        wait_idx(1)
        fire_g(1, 1)
        drain_g(0)
        fire_s(0, 0)
        loadidx(3, 3)

        def body(c, carry):
            b = lax.rem(c, 2)
            pb = lax.rem(c + 1, 2)
            b4 = lax.rem(c, 4)
            wait_idx(b4)
            drain_s(b)
            fire_g(b4, b)
            drain_g(pb)
            fire_s(lax.rem(c + 3, 4), pb)

            @pl.when(c < nch - 2)
            def _():
                loadidx(c + 2, lax.rem(c + 2, 4))

            return carry

        lax.fori_loop(2, nch, body, 0)
        lastb = (nch - 1) % 2
        drain_g(lastb)
        fire_s((nch - 1) % 4, lastb)
        drain_s((nch - 2) % 2)
        drain_s(lastb)
        plsc.subcore_barrier()
        pltpu.sync_copy(
            acc.at[pl.ds(tid * per, per)],
            out_hbm.at[cid, pl.ds(tid * per, per)],
        )

        @pl.when(tid == 0)
        def _():
            pltpu.sync_copy(
                acc.at[pl.ds(NT * per, rem)],
                out_hbm.at[cid, pl.ds(NT * per, rem)],
            )

    return k(z_flat, src_r, dst_r)


BF = jnp.bfloat16


def _split(a):
    hi = a.astype(BF)
    return hi, (a - hi.astype(FP)).astype(BF)


def _dot3(a, b_hi, b_lo):
    a_hi, a_lo = _split(a)
    return (jnp.dot(a_hi, b_hi, preferred_element_type=FP)
            + jnp.dot(a_lo, b_hi, preferred_element_type=FP)
            + jnp.dot(a_hi, b_lo, preferred_element_type=FP))


def _sig(v):
    return 0.5 * jnp.tanh(0.5 * v) + 0.5


def _tc_lstm(x2, deg16, wrep_hi, wrep_lo, whh_hi, whh_lo, bsum, w1):
    grid = (NN // BN,)

    def body(x_ref, deg_ref, wrh_ref, wrl_ref, whh_h_ref, whh_l_ref, b_ref,
             w1_ref, z_ref, dis_ref):
        x = x_ref[...]
        g_all = _dot3(x, wrh_ref[...], wrl_ref[...])
        whh_h = whh_h_ref[...]
        whh_l = whh_l_ref[...]
        b = b_ref[...]
        h = jnp.zeros((BN, HID), dtype=FP)
        c = jnp.zeros((BN, HID), dtype=FP)
        for t in range(TT):
            g = g_all[:, t * 128:(t + 1) * 128] + b + _dot3(h, whh_h, whh_l)
            ig = _sig(g[:, 0:32])
            fg = _sig(g[:, 32:64])
            gg = jnp.tanh(g[:, 64:96])
            og = _sig(g[:, 96:128])
            c = fg * c + ig * gg
            h = og * jnp.tanh(c)
        dis = lax.rsqrt(deg_ref[0, :, 0:1] + deg_ref[1, :, 0:1] + 1.0)
        z = dis * jnp.dot(h, w1_ref[...], preferred_element_type=FP,
                          precision=lax.Precision.HIGHEST)
        z_ref[0] = z[:, :HHALF]
        z_ref[1] = z[:, HHALF:]
        dis_ref[...] = dis

    return pl.pallas_call(
        body,
        grid=grid,
        in_specs=[
            pl.BlockSpec((BN, TT * FF), lambda i: (i, 0)),
            pl.BlockSpec((NC, BN, HHALF), lambda i: (0, i, 0)),
            pl.BlockSpec((TT * FF, TT * 128), lambda i: (0, 0)),
            pl.BlockSpec((TT * FF, TT * 128), lambda i: (0, 0)),
            pl.BlockSpec((HID, 128), lambda i: (0, 0)),
            pl.BlockSpec((HID, 128), lambda i: (0, 0)),
            pl.BlockSpec((1, 128), lambda i: (0, 0)),
            pl.BlockSpec((HID, HID), lambda i: (0, 0)),
        ],
        out_specs=[
            pl.BlockSpec((NC, BN, HHALF), lambda i: (0, i, 0)),
            pl.BlockSpec((BN, 1), lambda i: (i, 0)),
        ],
        out_shape=[
            jax.ShapeDtypeStruct((NC, NN, HHALF), FP),
            jax.ShapeDtypeStruct((NN, 1), FP),
        ],
    )(x2, deg16, wrep_hi, wrep_lo, whh_hi, whh_lo, bsum, w1)


def _tc_glue(acc, dis, b, w):
    grid = (NN // BN,)

    def body(acc_ref, dis_ref, b_ref, w_ref, z_ref):
        a = jnp.concatenate([acc_ref[0], acc_ref[1]], axis=1)
        dis = dis_ref[...]
        hcur = jnp.maximum(a * dis + b_ref[...], 0.0)
        z = dis * jnp.dot(hcur, w_ref[...], preferred_element_type=FP,
                          precision=lax.Precision.HIGHEST)
        z_ref[0] = z[:, :HHALF]
        z_ref[1] = z[:, HHALF:]

    return pl.pallas_call(
        body,
        grid=grid,
        in_specs=[
            pl.BlockSpec((NC, BN, HHALF), lambda i: (0, i, 0)),
            pl.BlockSpec((BN, 1), lambda i: (i, 0)),
            pl.BlockSpec((1, HID), lambda i: (0, 0)),
            pl.BlockSpec((HID, HID), lambda i: (0, 0)),
        ],
        out_specs=pl.BlockSpec((NC, BN, HHALF), lambda i: (0, i, 0)),
        out_shape=jax.ShapeDtypeStruct((NC, NN, HHALF), FP),
    )(acc, dis, b, w)


def _tc_head(acc, dis, b3, wh1, bh1, wh2row, bh2):
    grid = (NN // BN,)

    def body(acc_ref, dis_ref, b3_ref, wh1_ref, bh1_ref, wh2_ref, bh2_ref, o_ref):
        a = jnp.concatenate([acc_ref[0], acc_ref[1]], axis=1)
        h3 = a * dis_ref[...] + b3_ref[...]
        hid = jnp.maximum(
            jnp.dot(h3, wh1_ref[...], preferred_element_type=FP,
                    precision=lax.Precision.HIGHEST) + bh1_ref[...], 0.0)
        o = jnp.sum(hid * wh2_ref[...], axis=1, keepdims=True) + bh2_ref[...]
        o_ref[...] = jax.nn.sigmoid(o)

    return pl.pallas_call(
        body,
        grid=grid,
        in_specs=[
            pl.BlockSpec((NC, BN, HHALF), lambda i: (0, i, 0)),
            pl.BlockSpec((BN, 1), lambda i: (i, 0)),
            pl.BlockSpec((1, HID), lambda i: (0, 0)),
            pl.BlockSpec((HID, HHALF), lambda i: (0, 0)),
            pl.BlockSpec((1, HHALF), lambda i: (0, 0)),
            pl.BlockSpec((1, HHALF), lambda i: (0, 0)),
            pl.BlockSpec((1, 1), lambda i: (0, 0)),
        ],
        out_specs=pl.BlockSpec((BN, 1), lambda i: (i, 0)),
        out_shape=jax.ShapeDtypeStruct((NN, 1), FP),
    )(acc, dis, b3, wh1, bh1, wh2row, bh2)


def kernel(x_seq, edge_index, W_ih, W_hh, b_ih, b_hh, W1, b1, W2, b2, W3, b3,
           Wh1, bh1, Wh2, bh2):
    e = edge_index.shape[1]
    d_pt = -(-e // (NT * DESC))
    d_pt = -(-d_pt // (NC * JB)) * (NC * JB)
    e_pad = NT * d_pt * DESC
    pad = e_pad - e
    src = jnp.concatenate([edge_index[0], jnp.zeros((pad,), jnp.int32)])
    dst = jnp.concatenate([edge_index[1], jnp.full((pad,), NN, jnp.int32)])
    dst_r = dst.reshape(NT, d_pt, DESC)
    src_r = src.reshape(NT, d_pt, DESC)

    wrep = jnp.zeros((TT * FF, TT * 128), FP)
    wih_t = W_ih.T
    for t in range(TT):
        wrep = wrep.at[t * FF:(t + 1) * FF, t * 128:(t + 1) * 128].set(wih_t)
    whh_t = W_hh.T
    wrep_hi, wrep_lo = _split(wrep)
    whh_hi, whh_lo = _split(whh_t)
    bsum = (b_ih + b_hh).reshape(1, 128)
    x2 = x_seq.reshape(NN, TT * FF)

    zeros_h = jnp.zeros((NN // NT, HHALF), FP)
    ones_h = jnp.ones((DESC, HHALF), FP)

    deg2 = _sc_degree(dst_r, zeros_h, ones_h)
    z1, dis = _tc_lstm(x2, deg2, wrep_hi, wrep_lo, whh_hi, whh_lo, bsum, W1)

    acc1 = _sc_propagate(z1.reshape(NC * NN, HHALF), src_r, dst_r)
    z2 = _tc_glue(acc1, dis, b1.reshape(1, HID), W2)
    acc2 = _sc_propagate(z2.reshape(NC * NN, HHALF), src_r, dst_r)
    z3 = _tc_glue(acc2, dis, b2.reshape(1, HID), W3)
    acc3 = _sc_propagate(z3.reshape(NC * NN, HHALF), src_r, dst_r)

    out = _tc_head(acc3, dis, b3.reshape(1, HID), Wh1, bh1.reshape(1, HHALF),
                   Wh2.reshape(1, HHALF), bh2.reshape(1, 1))
    return out.reshape(NN)


# speedup vs baseline: 19.2641x; 1.2232x over previous
import functools

import jax
import jax.numpy as jnp
from jax import lax
from jax.experimental import pallas as pl
from jax.experimental.pallas import tpu as pltpu
from jax.experimental.pallas import tpu_sc as plsc

NN = 100000
NP = 100352
TT = 12
FF = 9
HID = 32
NT = 16
NC = 2
DESC = 128
JB = 5
HHALF = 16
N_ACC = NN + 8
BN = 2048
PBN = BN // 8
PR = NP // 8
GRID = NP // BN
FP = jnp.float32
BF = jnp.bfloat16


def _mesh():
    return plsc.VectorSubcoreMesh(core_axis_name="c", subcore_axis_name="s")


def _tile_rows(r):
    per = (r // NT) // 8 * 8
    rem = r - NT * per
    assert rem % 8 == 0
    return per, rem


def _sc_degree(dstm, dstp, zeros_h, ones_h, d_pt):
    rows_main = dstm.shape[0]
    half_pt = d_pt // NC
    nch = half_pt // JB
    zper, zrem = _tile_rows(NN)

    @functools.partial(
        pl.kernel,
        out_type=jax.ShapeDtypeStruct((NC, NP, HHALF), FP),
        mesh=_mesh(),
        compiler_params=pltpu.CompilerParams(use_tc_tiling_on_sc=False),
        scratch_types=[
            pltpu.VMEM_SHARED((N_ACC, HHALF), FP),
            pltpu.VMEM((JB, DESC), jnp.int32),
            pltpu.VMEM((DESC, HHALF), FP),
        ],
    )
    def k(dstm_hbm, dstp_hbm, zeros_hbm, ones_hbm, out_hbm, acc, dst_v, ones_v):
        cid = lax.axis_index("c")
        tid = lax.axis_index("s")
        pltpu.sync_copy(zeros_hbm.at[pl.ds(0, zper)],
                        acc.at[pl.ds(tid * zper, zper)])

        @pl.when(tid == 0)
        def _():
            pltpu.sync_copy(zeros_hbm.at[pl.ds(0, zrem)],
                            acc.at[pl.ds(NT * zper, zrem)])

        pltpu.sync_copy(ones_hbm, ones_v)
        plsc.subcore_barrier()
        base = tid * d_pt + cid * half_pt

        def chunk(ci, carry):
            r = base + ci * JB

            @pl.when(r < rows_main)
            def _():
                pltpu.sync_copy(dstm_hbm.at[pl.ds(r, JB)], dst_v)

            @pl.when(r >= rows_main)
            def _():
                pltpu.sync_copy(dstp_hbm.at[pl.ds(r - rows_main, JB)], dst_v)

            for j in range(JB):
                pltpu.sync_copy(ones_v, acc.at[dst_v.at[j]], add=True)
            return carry

        lax.fori_loop(0, nch, chunk, 0)
        plsc.subcore_barrier()
        pltpu.sync_copy(acc.at[pl.ds(tid * zper, zper)],
                        out_hbm.at[cid, pl.ds(tid * zper, zper)])

        @pl.when(tid == 0)
        def _():
            pltpu.sync_copy(acc.at[pl.ds(NT * zper, zrem)],
                            out_hbm.at[cid, pl.ds(NT * zper, zrem)])

    return k(dstm, dstp, zeros_h, ones_h)


def _sc_propagate(z_flat, srcm, srcp, dstm, dstp, d_pt):
    rows_main = dstm.shape[0]
    nch = d_pt // JB
    per, rem = _tile_rows(NN)
    ch = JB * DESC

    @functools.partial(
        pl.kernel,
        out_type=jax.ShapeDtypeStruct((NC, NP, HHALF), FP),
        mesh=_mesh(),
        compiler_params=pltpu.CompilerParams(use_tc_tiling_on_sc=False),
        scratch_types=[
            pltpu.VMEM_SHARED((N_ACC, HHALF), FP),
            pltpu.VMEM((4, JB, DESC), jnp.int32),
            pltpu.VMEM((4, JB, DESC), jnp.int32),
            pltpu.VMEM((2, ch, HHALF), FP),
            pltpu.SemaphoreType.DMA((2,)),
            pltpu.SemaphoreType.DMA((4,)),
            pltpu.SemaphoreType.DMA((2,)),
        ],
    )
    def k(z_hbm, srcm_hbm, srcp_hbm, dstm_hbm, dstp_hbm, out_hbm, acc,
          src_v, dst_v, rows_v, gsem, isem, ssem):
        cid = lax.axis_index("c")
        tid = lax.axis_index("s")
        zc = z_hbm.at[pl.ds(cid * NP, NP)]
        pltpu.sync_copy(zc.at[pl.ds(tid * per, per)], acc.at[pl.ds(tid * per, per)])

        @pl.when(tid == 0)
        def _():
            pltpu.sync_copy(zc.at[pl.ds(NT * per, rem)], acc.at[pl.ds(NT * per, rem)])

        plsc.subcore_barrier()

        def loadidx(c, b4):
            r = tid * d_pt + c * JB

            @pl.when(r < rows_main)
            def _():
                pltpu.async_copy(srcm_hbm.at[pl.ds(r, JB)], src_v.at[b4],
                                 isem.at[b4])
                pltpu.async_copy(dstm_hbm.at[pl.ds(r, JB)], dst_v.at[b4],
                                 isem.at[b4])

            @pl.when(r >= rows_main)
            def _():
                rp = r - rows_main
                pltpu.async_copy(srcp_hbm.at[pl.ds(rp, JB)], src_v.at[b4],
                                 isem.at[b4])
                pltpu.async_copy(dstp_hbm.at[pl.ds(rp, JB)], dst_v.at[b4],
                                 isem.at[b4])

        def wait_idx(b4):
            pltpu.make_async_copy(srcm_hbm.at[pl.ds(0, JB)],
                                  src_v.at[b4], isem.at[b4]).wait()
            pltpu.make_async_copy(dstm_hbm.at[pl.ds(0, JB)],
                                  dst_v.at[b4], isem.at[b4]).wait()

        def fire_g(b4, b):
            for j in range(JB):
                pltpu.async_copy(zc.at[src_v.at[b4, j]],
                                 rows_v.at[b, pl.ds(j * DESC, DESC)], gsem.at[b])

        def drain_g(b):
            pltpu.make_async_copy(z_hbm.at[pl.ds(0, ch)], rows_v.at[b],
                                  gsem.at[b]).wait()

        def fire_s(b4, b):
            for j in range(JB):
                pltpu.async_copy(rows_v.at[b, pl.ds(j * DESC, DESC)],
                                 acc.at[dst_v.at[b4, j]], ssem.at[b], add=True)

        def drain_s(b):
            pltpu.make_async_copy(rows_v.at[b], acc.at[pl.ds(0, ch)],
                                  ssem.at[b]).wait()

        loadidx(0, 0)
        loadidx(1, 1)
        loadidx(2, 2)
        wait_idx(0)
        fire_g(0, 0)
        wait_idx(1)
        fire_g(1, 1)
        drain_g(0)
        fire_s(0, 0)
        loadidx(3, 3)

        def body(c, carry):
            b = lax.rem(c, 2)
            pb = lax.rem(c + 1, 2)
            b4 = lax.rem(c, 4)
            wait_idx(b4)
            drain_s(b)
            fire_g(b4, b)
            drain_g(pb)
            fire_s(lax.rem(c + 3, 4), pb)

            @pl.when(c < nch - 2)
            def _():
                loadidx(c + 2, lax.rem(c + 2, 4))

            return carry

        lax.fori_loop(2, nch, body, 0)
        lastb = (nch - 1) % 2
        drain_g(lastb)
        fire_s((nch - 1) % 4, lastb)
        drain_s((nch - 2) % 2)
        drain_s(lastb)
        plsc.subcore_barrier()
        pltpu.sync_copy(
            acc.at[pl.ds(tid * per, per)],
            out_hbm.at[cid, pl.ds(tid * per, per)],
        )

        @pl.when(tid == 0)
        def _():
            pltpu.sync_copy(
                acc.at[pl.ds(NT * per, rem)],
                out_hbm.at[cid, pl.ds(NT * per, rem)],
            )

    return k(z_flat, srcm, srcp, dstm, dstp)


def _split(a):
    hi = a.astype(BF)
    return hi, (a - hi.astype(FP)).astype(BF)


def _dot3(a, b_hi, b_lo):
    a_hi, a_lo = _split(a)
    return (jnp.dot(a_hi, b_hi, preferred_element_type=FP)
            + jnp.dot(a_lo, b_hi, preferred_element_type=FP)
            + jnp.dot(a_hi, b_lo, preferred_element_type=FP))


def _sig(v):
    return 0.5 * jnp.tanh(0.5 * v) + 0.5


def _tc_lstm(x2, deg2, wrep_hi, wrep_lo, whh_hi, whh_lo, bsum, w1):
    lbn = 2000
    grid = (NN // lbn,)

    def body(x_ref, deg_ref, wrh_ref, wrl_ref, whh_h_ref, whh_l_ref, b_ref,
             w1_ref, z_ref):
        x = x_ref[...]
        g_all = _dot3(x, wrh_ref[...], wrl_ref[...])
        whh_h = whh_h_ref[...]
        whh_l = whh_l_ref[...]
        b = b_ref[...]
        h = jnp.zeros((lbn, HID), dtype=FP)
        c = jnp.zeros((lbn, HID), dtype=FP)
        for t in range(TT):
            g = g_all[:, t * 128:(t + 1) * 128] + b + _dot3(h, whh_h, whh_l)
            ig = _sig(g[:, 0:32])
            fg = _sig(g[:, 32:64])
            gg = jnp.tanh(g[:, 64:96])
            og = _sig(g[:, 96:128])
            c = fg * c + ig * gg
            h = og * jnp.tanh(c)
        dis = lax.rsqrt(deg_ref[0, :, 0:1] + deg_ref[1, :, 0:1] + 1.0)
        z = dis * jnp.dot(h, w1_ref[...], preferred_element_type=FP,
                          precision=lax.Precision.HIGHEST)
        z_ref[0] = z[:, :HHALF]
        z_ref[1] = z[:, HHALF:]

    return pl.pallas_call(
        body,
        grid=grid,
        in_specs=[
            pl.BlockSpec((lbn, TT * FF), lambda i: (i, 0)),
            pl.BlockSpec((NC, lbn, HHALF), lambda i: (0, i, 0)),
            pl.BlockSpec((TT * FF, TT * 128), lambda i: (0, 0)),
            pl.BlockSpec((TT * FF, TT * 128), lambda i: (0, 0)),
            pl.BlockSpec((HID, 128), lambda i: (0, 0)),
            pl.BlockSpec((HID, 128), lambda i: (0, 0)),
            pl.BlockSpec((1, 128), lambda i: (0, 0)),
            pl.BlockSpec((HID, HID), lambda i: (0, 0)),
        ],
        out_specs=pl.BlockSpec((NC, lbn, HHALF), lambda i: (0, i, 0)),
        out_shape=jax.ShapeDtypeStruct((NC, NP, HHALF), FP),
    )(x2, deg2, wrep_hi, wrep_lo, whh_hi, whh_lo, bsum, w1)


def _tc_glue(accP, degP, b0p, b1p, w00, w01, w10, w11):
    def body(acc_ref, deg_ref, b0_ref, b1_ref, w00_ref, w01_ref, w10_ref,
             w11_ref, z_ref):
        disp = lax.rsqrt(deg_ref[0] + deg_ref[1] + 1.0)
        h0 = jnp.maximum(acc_ref[0] * disp + b0_ref[...], 0.0)
        h1 = jnp.maximum(acc_ref[1] * disp + b1_ref[...], 0.0)

        def dt(a, w_ref):
            return jnp.dot(a, w_ref[...], preferred_element_type=FP,
                           precision=lax.Precision.HIGHEST)

        z_ref[0] = disp * (dt(h0, w00_ref) + dt(h1, w10_ref))
        z_ref[1] = disp * (dt(h0, w01_ref) + dt(h1, w11_ref))

    wspec = pl.BlockSpec((128, 128), lambda i: (0, 0))
    bspec = pl.BlockSpec((1, 128), lambda i: (0, 0))
    pspec = pl.BlockSpec((NC, PBN, 128), lambda i: (0, i, 0))
    return pl.pallas_call(
        body,
        grid=(GRID,),
        in_specs=[pspec, pspec, bspec, bspec, wspec, wspec, wspec, wspec],
        out_specs=pspec,
        out_shape=jax.ShapeDtypeStruct((NC, PR, 128), FP),
    )(accP, degP, b0p, b1p, w00, w01, w10, w11)


def _tc_head(accP, degP, b0p, b1p, wh0, wh1bd, bh1p, sel, bh2):
    def body(acc_ref, deg_ref, b0_ref, b1_ref, wh0_ref, wh1_ref, bh1_ref,
             sel_ref, bh2_ref, o_ref):
        disp = lax.rsqrt(deg_ref[0] + deg_ref[1] + 1.0)
        h0 = acc_ref[0] * disp + b0_ref[...]
        h1 = acc_ref[1] * disp + b1_ref[...]
        hid = jnp.maximum(
            jnp.dot(h0, wh0_ref[...], preferred_element_type=FP,
                    precision=lax.Precision.HIGHEST)
            + jnp.dot(h1, wh1_ref[...], preferred_element_type=FP,
                      precision=lax.Precision.HIGHEST)
            + bh1_ref[...], 0.0)
        o = jnp.dot(hid, sel_ref[...], preferred_element_type=FP,
                    precision=lax.Precision.HIGHEST) + bh2_ref[...]
        o_ref[...] = jax.nn.sigmoid(o)

    wspec = pl.BlockSpec((128, 128), lambda i: (0, 0))
    bspec = pl.BlockSpec((1, 128), lambda i: (0, 0))
    pspec = pl.BlockSpec((NC, PBN, 128), lambda i: (0, i, 0))
    return pl.pallas_call(
        body,
        grid=(GRID,),
        in_specs=[pspec, pspec, bspec, bspec, wspec, wspec, bspec,
                  pl.BlockSpec((128, 8), lambda i: (0, 0)),
                  pl.BlockSpec((1, 8), lambda i: (0, 0))],
        out_specs=pl.BlockSpec((PBN, 8), lambda i: (i, 0)),
        out_shape=jax.ShapeDtypeStruct((PR, 8), FP),
    )(accP, degP, b0p, b1p, wh0, wh1bd, bh1p, sel, bh2)


def _bd(m):
    z = jnp.zeros((128, 128), FP)
    for s in range(8):
        z = z.at[s * 16:(s + 1) * 16, s * 16:(s + 1) * 16].set(m)
    return z


def _tile8(v):
    return jnp.tile(v.reshape(1, HHALF), (1, 8)).reshape(1, 128)


def kernel(x_seq, edge_index, W_ih, W_hh, b_ih, b_hh, W1, b1, W2, b2, W3, b3,
           Wh1, bh1, Wh2, bh2):
    e = edge_index.shape[1]
    assert e % DESC == 0
    rows_main = e // DESC
    assert rows_main % JB == 0
    d_pt = -(-rows_main // NT)
    d_pt = -(-d_pt // (NC * JB)) * (NC * JB)
    padr = NT * d_pt - rows_main

    srcm = edge_index[0].reshape(rows_main, DESC)
    dstm = edge_index[1].reshape(rows_main, DESC)
    srcp = jnp.zeros((padr, DESC), jnp.int32)
    dstp = jnp.full((padr, DESC), NN, jnp.int32)

    wrep = jnp.zeros((TT * FF, TT * 128), FP)
    wih_t = W_ih.T
    for t in range(TT):
        wrep = wrep.at[t * FF:(t + 1) * FF, t * 128:(t + 1) * 128].set(wih_t)
    whh_t = W_hh.T
    wrep_hi, wrep_lo = _split(wrep)
    whh_hi, whh_lo = _split(whh_t)
    bsum = (b_ih + b_hh).reshape(1, 128)
    x2 = x_seq.reshape(NN, TT * FF)

    zeros_h = jnp.zeros((NN // NT, HHALF), FP)
    ones_h = jnp.ones((DESC, HHALF), FP)

    def quads(w):
        return (_bd(w[:16, :16]), _bd(w[:16, 16:]),
                _bd(w[16:, :16]), _bd(w[16:, 16:]))

    w2q = quads(W2)
    w3q = quads(W3)
    wh0 = _bd(Wh1[:16, :])
    wh1bd = _bd(Wh1[16:, :])
    sel = jnp.zeros((128, 8), FP)
    for s in range(8):
        sel = sel.at[s * 16:(s + 1) * 16, s].set(Wh2[:, 0])

    deg2 = _sc_degree(dstm, dstp, zeros_h, ones_h, d_pt)
    degP = deg2.reshape(NC, PR, 128)

    z1 = _tc_lstm(x2, deg2, wrep_hi, wrep_lo, whh_hi, whh_lo, bsum, W1)
    acc1 = _sc_propagate(z1.reshape(NC * NP, HHALF), srcm, srcp, dstm, dstp, d_pt)

    z2 = _tc_glue(acc1.reshape(NC, PR, 128), degP,
                  _tile8(b1[:16]), _tile8(b1[16:]), *w2q)
    acc2 = _sc_propagate(z2.reshape(NC * NP, HHALF), srcm, srcp, dstm, dstp, d_pt)

    z3 = _tc_glue(acc2.reshape(NC, PR, 128), degP,
                  _tile8(b2[:16]), _tile8(b2[16:]), *w3q)
    acc3 = _sc_propagate(z3.reshape(NC * NP, HHALF), srcm, srcp, dstm, dstp, d_pt)

    out = _tc_head(acc3.reshape(NC, PR, 128), degP,
                   _tile8(b3[:16]), _tile8(b3[16:]), wh0, wh1bd,
                   _tile8(bh1), sel, bh2.reshape(1, 1) * jnp.ones((1, 8), FP))
    return out.reshape(NP)[:NN]
